# baseline (speedup 1.0000x reference)
import dataclasses
import functools

import jax
import jax.numpy as jnp
import numpy as np
from jax import lax
from jax.experimental import pallas as pl
from jax.experimental.pallas import tpu as pltpu
from jax.experimental.pallas import tpu_sc as plsc

NH, NL, NP = 8, 4, 4
SPATIAL = np.array([[64, 64], [32, 32], [16, 16], [8, 8]], dtype=np.int64)
START = np.array([0, 4096, 5120, 5376], dtype=np.int64)
LV = int((SPATIAL[:, 0] * SPATIAL[:, 1]).sum())
BS, LQ, D = 4, 900, 256
HD = D // NH
BQ = BS * LQ
R = BQ * NH
NCORN = 4
J = NL * NP * NCORN

_HIGH = jax.lax.Precision.HIGHEST

_LANE = np.arange(NH * NL * NP)
_L_OF = (_LANE // NP) % NL
_LW = SPATIAL[_L_OF, 1].astype(np.float32)
_LH = SPATIAL[_L_OF, 0].astype(np.float32)
_LST = START[_L_OF].astype(np.int32)
_LHD = (_LANE // (NL * NP)).astype(np.int32)
_G = np.kron(np.eye(NH, dtype=np.float32), np.ones((NL * NP, NL * NP), np.float32))


def _vproj_body(x_ref, w_ref, b_ref, o_ref):
    o_ref[...] = (jnp.dot(x_ref[...].astype(jnp.bfloat16),
                          w_ref[...].astype(jnp.bfloat16),
                          preferred_element_type=jnp.float32)
                  + b_ref[...]).astype(jnp.bfloat16)


def _params_body(q_ref, wox_ref, woy_ref, wat_ref, box_ref, boy_ref, bat_ref,
                 g_ref, rpx_ref, rpy_ref, brow_ref, lw_ref, lh_ref, lst_ref,
                 lhd_ref,
                 i00_ref, i10_ref, i01_ref, i11_ref,
                 w00_ref, w10_ref, w01_ref, w11_ref):
    q = q_ref[...].astype(jnp.bfloat16)
    att = jnp.exp(jnp.dot(q, wat_ref[...].astype(jnp.bfloat16),
                          preferred_element_type=jnp.float32) + bat_ref[...])
    s = jnp.dot(att, g_ref[...], preferred_element_type=jnp.float32)
    aw = att / s

    offx = jnp.dot(q, wox_ref[...].astype(jnp.bfloat16),
                   preferred_element_type=jnp.float32) + box_ref[...]
    offy = jnp.dot(q, woy_ref[...].astype(jnp.bfloat16),
                   preferred_element_type=jnp.float32) + boy_ref[...]
    W = lw_ref[...]
    H = lh_ref[...]
    x = (rpx_ref[...] + offx / W) * W - 0.5
    y = (rpy_ref[...] + offy / H) * H - 0.5
    x0 = jnp.floor(x)
    y0 = jnp.floor(y)
    wx = x - x0
    wy = y - y0
    x0i = x0.astype(jnp.int32)
    y0i = y0.astype(jnp.int32)
    wi = W.astype(jnp.int32)
    hi = H.astype(jnp.int32)
    base = brow_ref[...] + lhd_ref[...]
    lst = lst_ref[...]

    def corner(xi, yi, bw, i_ref, w_ref_):
        valid = (xi >= 0) & (xi < wi) & (yi >= 0) & (yi < hi)
        xc = jnp.clip(xi, 0, wi - 1)
        yc = jnp.clip(yi, 0, hi - 1)
        pos = lst + yc * wi + xc
        i_ref[...] = base + pos * NH
        w_ref_[...] = aw * bw * valid.astype(jnp.float32)

    corner(x0i, y0i, (1 - wx) * (1 - wy), i00_ref, w00_ref)
    corner(x0i + 1, y0i, wx * (1 - wy), i10_ref, w10_ref)
    corner(x0i, y0i + 1, (1 - wx) * wy, i01_ref, w01_ref)
    corner(x0i + 1, y0i + 1, wx * wy, i11_ref, w11_ref)


def _outproj_body(x_ref, w_ref, b_ref, o_ref):
    o_ref[...] = jnp.dot(x_ref[...].astype(jnp.bfloat16),
                         w_ref[...].astype(jnp.bfloat16),
                         preferred_element_type=jnp.float32) + b_ref[...]


def _vproj(value2d, Wv, bv):
    blk = 640
    n = value2d.shape[0] // blk
    return pl.pallas_call(
        _vproj_body,
        grid=(n,),
        in_specs=[
            pl.BlockSpec((blk, D), lambda i: (i, 0)),
            pl.BlockSpec((D, D), lambda i: (0, 0)),
            pl.BlockSpec((1, D), lambda i: (0, 0)),
        ],
        out_specs=pl.BlockSpec((blk, D), lambda i: (i, 0)),
        out_shape=jax.ShapeDtypeStruct((value2d.shape[0], D), jnp.bfloat16),
    )(value2d, Wv, bv.reshape(1, D))


def _params(q2d, Wox, Woy, box, boy, Wattn, battn, rpx, rpy, brow):
    blk = 600
    n = BQ // blk
    L = NH * NL * NP
    rep = lambda shape: pl.BlockSpec(shape, lambda i: (0, 0))
    per = lambda shape: pl.BlockSpec(shape, lambda i: (i, 0))
    outs = [jax.ShapeDtypeStruct((BQ, L), jnp.int32)] * 4 + \
           [jax.ShapeDtypeStruct((BQ, L), jnp.float32)] * 4
    return pl.pallas_call(
        _params_body,
        grid=(n,),
        in_specs=[
            per((blk, D)),
            rep((D, L)), rep((D, L)), rep((D, L)),
            rep((1, L)), rep((1, L)), rep((1, L)),
            rep((L, L)),
            per((blk, L)), per((blk, L)),
            per((blk, L)),
            rep((1, L)), rep((1, L)), rep((1, L)), rep((1, L)),
        ],
        out_specs=[per((blk, L))] * 8,
        out_shape=outs,
    )(q2d, Wox, Woy, Wattn, box.reshape(1, L), boy.reshape(1, L),
      battn.reshape(1, L), jnp.asarray(_G), rpx, rpy, brow,
      jnp.asarray(_LW).reshape(1, L), jnp.asarray(_LH).reshape(1, L),
      jnp.asarray(_LST).reshape(1, L), jnp.asarray(_LHD).reshape(1, L))


def _outproj(x2d, Wout, bout):
    blk = 600
    n = BQ // blk
    return pl.pallas_call(
        _outproj_body,
        grid=(n,),
        in_specs=[
            pl.BlockSpec((blk, D), lambda i: (i, 0)),
            pl.BlockSpec((D, D), lambda i: (0, 0)),
            pl.BlockSpec((1, D), lambda i: (0, 0)),
        ],
        out_specs=pl.BlockSpec((blk, D), lambda i: (i, 0)),
        out_shape=jax.ShapeDtypeStruct((BQ, D), jnp.float32),
    )(x2d, Wout, bout.reshape(1, D))


_NW = 32
_RPW = R // _NW
_CH = 30
_NCHUNK = _RPW // _CH
_IPC = _CH * NL * NP
_SUBS = [(s, min(128, _IPC - s)) for s in range(0, _IPC, 128)]


_SC_PARAMS = pltpu.CompilerParams(use_tc_tiling_on_sc=False)
if "needs_layout_passes" in pltpu.CompilerParams.__dataclass_fields__:
    _SC_PARAMS = dataclasses.replace(_SC_PARAMS, needs_layout_passes=False)


def _sc_sample(table, idxs, wgts):
    mesh = plsc.VectorSubcoreMesh(core_axis_name="c", subcore_axis_name="s")

    @functools.partial(
        pl.kernel,
        out_type=jax.ShapeDtypeStruct((R * HD,), jnp.float32),
        mesh=mesh,
        compiler_params=_SC_PARAMS,
        scratch_types=[
            pltpu.VMEM((3, NCORN * _IPC), jnp.int32),
            pltpu.VMEM((3, NCORN * _IPC), jnp.float32),
            pltpu.VMEM((3, NCORN * _IPC, HD), jnp.bfloat16),
            pltpu.VMEM((2, _CH * HD), jnp.float32),
            pltpu.SemaphoreType.DMA,
            pltpu.SemaphoreType.DMA,
            pltpu.SemaphoreType.DMA,
            pltpu.SemaphoreType.DMA,
            pltpu.SemaphoreType.DMA,
            pltpu.SemaphoreType.DMA,
            pltpu.SemaphoreType.DMA,
            pltpu.SemaphoreType.DMA,
        ],
    )
    def sc_kernel(table_hbm, i0_hbm, i1_hbm, i2_hbm, i3_hbm,
                  w0_hbm, w1_hbm, w2_hbm, w3_hbm, out_hbm,
                  idx_v, w_v, rows_v, out_v,
                  sem_io0, sem_io1, sem_io2, sem_g0, sem_g1, sem_g2,
                  sem_o0, sem_o1):
        sem_io = [sem_io0, sem_io1, sem_io2]
        sem_g = [sem_g0, sem_g1, sem_g2]
        sem_o = [sem_o0, sem_o1]
        wid = lax.axis_index("s") * 2 + lax.axis_index("c")
        base0 = wid * _RPW
        ihs = [i0_hbm, i1_hbm, i2_hbm, i3_hbm]
        whs = [w0_hbm, w1_hbm, w2_hbm, w3_hbm]

        def load_idx(ci, b):
            o16 = (base0 + ci * _CH) * (NL * NP)
            for c in range(NCORN):
                pltpu.async_copy(ihs[c].at[pl.ds(o16, _IPC)],
                                 idx_v.at[b, pl.ds(c * _IPC, _IPC)], sem_io[b])
                pltpu.async_copy(whs[c].at[pl.ds(o16, _IPC)],
                                 w_v.at[b, pl.ds(c * _IPC, _IPC)], sem_io[b])

        def drain_idx(b):
            pltpu.make_async_copy(i0_hbm.at[pl.ds(0, NCORN * _IPC)],
                                  idx_v.at[b], sem_io[b]).wait()
            pltpu.make_async_copy(w0_hbm.at[pl.ds(0, NCORN * _IPC)],
                                  w_v.at[b], sem_io[b]).wait()

        def fire_gathers(b):
            for c in range(NCORN):
                for (s, n) in _SUBS:
                    pltpu.async_copy(
                        table_hbm.at[idx_v.at[b].at[pl.ds(c * _IPC + s, n)]],
                        rows_v.at[b].at[pl.ds(c * _IPC + s, n)], sem_g[b])

        def drain_gathers(b):
            pltpu.make_async_copy(table_hbm.at[pl.ds(0, NCORN * _IPC)],
                                  rows_v.at[b], sem_g[b]).wait()

        def compute(ci, b, ob):
            @pl.loop(0, _CH)
            def _row(r):
                lin0 = r * (NL * NP)
                accs = []
                for c in range(NCORN):
                    a0 = jnp.zeros((16,), jnp.float32)
                    a1 = jnp.zeros((16,), jnp.float32)
                    wv = w_v[b, pl.ds(c * _IPC + lin0, 16)]
                    for u in range(16):
                        lin = c * _IPC + lin0 + u
                        wj = wv[u]
                        ev, od = plsc.unpack(rows_v[b, lin, :],
                                             format=plsc.PackFormat.INTERLEAVED)
                        a0 = a0 + wj * ev
                        a1 = a1 + wj * od
                    accs.append((a0, a1))
                acc0 = (accs[0][0] + accs[1][0]) + (accs[2][0] + accs[3][0])
                acc1 = (accs[0][1] + accs[1][1]) + (accs[2][1] + accs[3][1])
                out_v[ob, pl.ds(r * HD, 16)] = acc0
                out_v[ob, pl.ds(r * HD + 16, 16)] = acc1

            pltpu.async_copy(
                out_v.at[ob],
                out_hbm.at[pl.ds((base0 + ci * _CH) * HD, _CH * HD)],
                sem_o[ob])

        load_idx(0, 0)
        drain_idx(0)
        fire_gathers(0)
        load_idx(1, 1)
        load_idx(2, 2)
        drain_idx(1)
        fire_gathers(1)

        @pl.loop(0, _NCHUNK, step=3)
        def _pipe(ci):
            for b in (0, 1, 2):
                cur = ci + b
                bp2 = (b + 2) % 3
                drain_gathers(b)

                @pl.when(cur + 2 < _NCHUNK)
                def _():
                    drain_idx(bp2)
                    fire_gathers(bp2)

                for ob in (0, 1):
                    @pl.when((cur >= 2) & (cur % 2 == ob))
                    def _():
                        pltpu.make_async_copy(
                            out_v.at[ob], out_hbm.at[pl.ds(0, _CH * HD)],
                            sem_o[ob]).wait()

                for ob in (0, 1):
                    @pl.when(cur % 2 == ob)
                    def _():
                        compute(cur, b, ob)

                @pl.when(cur + 3 < _NCHUNK)
                def _():
                    load_idx(cur + 3, b)

        for ob in (0, 1):
            pltpu.make_async_copy(out_v.at[ob],
                                  out_hbm.at[pl.ds(0, _CH * HD)],
                                  sem_o[ob]).wait()

    return sc_kernel(table, *idxs, *wgts)


def kernel(query, reference_points, value, value_spatial_shapes,
           value_level_start_index, Wv, bv, Woff, boff, Wattn, battn,
           Wout, bout):
    L = NH * NL * NP

    v2 = _vproj(value.reshape(BS * LV, D), Wv, bv)
    table = v2.reshape(BS * LV * NH, HD)

    q2d = query.reshape(BQ, D)
    Wox = Woff[:, 0::2]
    Woy = Woff[:, 1::2]
    box = boff[0::2]
    boy = boff[1::2]
    rpx = jnp.broadcast_to(reference_points[..., 0].reshape(BQ, 1, NL, 1),
                           (BQ, NH, NL, NP)).reshape(BQ, L)
    rpy = jnp.broadcast_to(reference_points[..., 1].reshape(BQ, 1, NL, 1),
                           (BQ, NH, NL, NP)).reshape(BQ, L)
    brow = jnp.broadcast_to(
        (jnp.repeat(jnp.arange(BS, dtype=jnp.int32) * (LV * NH), LQ)
         ).reshape(BQ, 1), (BQ, L))
    i00, i10, i01, i11, w00, w10, w01, w11 = _params(
        q2d, Wox, Woy, box, boy, Wattn, battn, rpx, rpy, brow)

    idxs = [a.reshape(-1) for a in (i00, i10, i01, i11)]
    wgts = [a.reshape(-1) for a in (w00, w10, w01, w11)]

    sampled = _sc_sample(table, idxs, wgts)

    perm32 = np.concatenate([np.arange(0, HD, 2), np.arange(1, HD, 2)])
    permg = (np.arange(D) // HD) * HD + perm32[np.arange(D) % HD]
    out = _outproj(sampled.reshape(BQ, D), Wout[jnp.asarray(permg), :], bout)
    return out.reshape(BS, LQ, D)


# speedup vs baseline: 1.0368x; 1.0368x over previous
import dataclasses
import functools

import jax
import jax.numpy as jnp
import numpy as np
from jax import lax
from jax.experimental import pallas as pl
from jax.experimental.pallas import tpu as pltpu
from jax.experimental.pallas import tpu_sc as plsc

NH, NL, NP = 8, 4, 4
SPATIAL = np.array([[64, 64], [32, 32], [16, 16], [8, 8]], dtype=np.int64)
START = np.array([0, 4096, 5120, 5376], dtype=np.int64)
LV = int((SPATIAL[:, 0] * SPATIAL[:, 1]).sum())
BS, LQ, D = 4, 900, 256
HD = D // NH
BQ = BS * LQ
R = BQ * NH
NCORN = 4
J = NL * NP * NCORN

_HIGH = jax.lax.Precision.HIGHEST

_LANE = np.arange(NH * NL * NP)
_L_OF = (_LANE // NP) % NL
_LW = SPATIAL[_L_OF, 1].astype(np.float32)
_LH = SPATIAL[_L_OF, 0].astype(np.float32)
_LST = START[_L_OF].astype(np.int32)
_LHD = (_LANE // (NL * NP)).astype(np.int32)
_G = np.kron(np.eye(NH, dtype=np.float32), np.ones((NL * NP, NL * NP), np.float32))


def _vproj_body(x_ref, w_ref, b_ref, o_ref):
    o_ref[...] = (jnp.dot(x_ref[...].astype(jnp.bfloat16),
                          w_ref[...].astype(jnp.bfloat16),
                          preferred_element_type=jnp.float32)
                  + b_ref[...]).astype(jnp.bfloat16)


def _params_body(q_ref, wox_ref, woy_ref, wat_ref, box_ref, boy_ref, bat_ref,
                 g_ref, rpx_ref, rpy_ref, brow_ref, lw_ref, lh_ref, lst_ref,
                 lhd_ref,
                 i00_ref, i10_ref, i01_ref, i11_ref,
                 w00_ref, w10_ref, w01_ref, w11_ref):
    q = q_ref[...].astype(jnp.bfloat16)
    att = jnp.exp(jnp.dot(q, wat_ref[...].astype(jnp.bfloat16),
                          preferred_element_type=jnp.float32) + bat_ref[...])
    s = jnp.dot(att, g_ref[...], preferred_element_type=jnp.float32)
    aw = att / s

    offx = jnp.dot(q, wox_ref[...].astype(jnp.bfloat16),
                   preferred_element_type=jnp.float32) + box_ref[...]
    offy = jnp.dot(q, woy_ref[...].astype(jnp.bfloat16),
                   preferred_element_type=jnp.float32) + boy_ref[...]
    W = lw_ref[...]
    H = lh_ref[...]
    x = (rpx_ref[...] + offx / W) * W - 0.5
    y = (rpy_ref[...] + offy / H) * H - 0.5
    x0 = jnp.floor(x)
    y0 = jnp.floor(y)
    wx = x - x0
    wy = y - y0
    x0i = x0.astype(jnp.int32)
    y0i = y0.astype(jnp.int32)
    wi = W.astype(jnp.int32)
    hi = H.astype(jnp.int32)
    base = brow_ref[...] + lhd_ref[...]
    lst = lst_ref[...]

    def corner(xi, yi, bw, i_ref, w_ref_):
        valid = (xi >= 0) & (xi < wi) & (yi >= 0) & (yi < hi)
        xc = jnp.clip(xi, 0, wi - 1)
        yc = jnp.clip(yi, 0, hi - 1)
        pos = lst + yc * wi + xc
        i_ref[...] = base + pos * NH
        w_ref_[...] = aw * bw * valid.astype(jnp.float32)

    corner(x0i, y0i, (1 - wx) * (1 - wy), i00_ref, w00_ref)
    corner(x0i + 1, y0i, wx * (1 - wy), i10_ref, w10_ref)
    corner(x0i, y0i + 1, (1 - wx) * wy, i01_ref, w01_ref)
    corner(x0i + 1, y0i + 1, wx * wy, i11_ref, w11_ref)


def _outproj_body(x_ref, w_ref, b_ref, o_ref):
    o_ref[...] = jnp.dot(x_ref[...].astype(jnp.bfloat16),
                         w_ref[...].astype(jnp.bfloat16),
                         preferred_element_type=jnp.float32) + b_ref[...]


def _vproj(value2d, Wv, bv):
    blk = 640
    n = value2d.shape[0] // blk
    return pl.pallas_call(
        _vproj_body,
        grid=(n,),
        in_specs=[
            pl.BlockSpec((blk, D), lambda i: (i, 0)),
            pl.BlockSpec((D, D), lambda i: (0, 0)),
            pl.BlockSpec((1, D), lambda i: (0, 0)),
        ],
        out_specs=pl.BlockSpec((blk, D), lambda i: (i, 0)),
        out_shape=jax.ShapeDtypeStruct((value2d.shape[0], D), jnp.bfloat16),
    )(value2d, Wv, bv.reshape(1, D))


def _params(q2d, Wox, Woy, box, boy, Wattn, battn, rpx, rpy, brow):
    blk = 600
    n = BQ // blk
    L = NH * NL * NP
    rep = lambda shape: pl.BlockSpec(shape, lambda i: (0, 0))
    per = lambda shape: pl.BlockSpec(shape, lambda i: (i, 0))
    outs = [jax.ShapeDtypeStruct((BQ, L), jnp.int32)] * 4 + \
           [jax.ShapeDtypeStruct((BQ, L), jnp.float32)] * 4
    return pl.pallas_call(
        _params_body,
        grid=(n,),
        in_specs=[
            per((blk, D)),
            rep((D, L)), rep((D, L)), rep((D, L)),
            rep((1, L)), rep((1, L)), rep((1, L)),
            rep((L, L)),
            per((blk, L)), per((blk, L)),
            per((blk, L)),
            rep((1, L)), rep((1, L)), rep((1, L)), rep((1, L)),
        ],
        out_specs=[per((blk, L))] * 8,
        out_shape=outs,
    )(q2d, Wox, Woy, Wattn, box.reshape(1, L), boy.reshape(1, L),
      battn.reshape(1, L), jnp.asarray(_G), rpx, rpy, brow,
      jnp.asarray(_LW).reshape(1, L), jnp.asarray(_LH).reshape(1, L),
      jnp.asarray(_LST).reshape(1, L), jnp.asarray(_LHD).reshape(1, L))


def _outproj(x2d, Wout, bout):
    blk = 600
    n = BQ // blk
    return pl.pallas_call(
        _outproj_body,
        grid=(n,),
        in_specs=[
            pl.BlockSpec((blk, D), lambda i: (i, 0)),
            pl.BlockSpec((D, D), lambda i: (0, 0)),
            pl.BlockSpec((1, D), lambda i: (0, 0)),
        ],
        out_specs=pl.BlockSpec((blk, D), lambda i: (i, 0)),
        out_shape=jax.ShapeDtypeStruct((BQ, D), jnp.float32),
    )(x2d, Wout, bout.reshape(1, D))


_NW = 32
_RPW = R // _NW
_CH = 45
_NCHUNK = _RPW // _CH
_IPC = _CH * NL * NP
_SUBS = [(s, min(128, _IPC - s)) for s in range(0, _IPC, 128)]


_SC_PARAMS = pltpu.CompilerParams(use_tc_tiling_on_sc=False)
if "needs_layout_passes" in pltpu.CompilerParams.__dataclass_fields__:
    _SC_PARAMS = dataclasses.replace(_SC_PARAMS, needs_layout_passes=False)


def _sc_sample(table, idxs, wgts):
    mesh = plsc.VectorSubcoreMesh(core_axis_name="c", subcore_axis_name="s")

    @functools.partial(
        pl.kernel,
        out_type=jax.ShapeDtypeStruct((R * HD,), jnp.float32),
        mesh=mesh,
        compiler_params=_SC_PARAMS,
        scratch_types=[
            pltpu.VMEM((2, NCORN * _IPC), jnp.int32),
            pltpu.VMEM((2, NCORN * _IPC), jnp.float32),
            pltpu.VMEM((2, NCORN * _IPC, HD), jnp.bfloat16),
            pltpu.VMEM((2, _CH * HD), jnp.float32),
            pltpu.SemaphoreType.DMA,
            pltpu.SemaphoreType.DMA,
            pltpu.SemaphoreType.DMA,
            pltpu.SemaphoreType.DMA,
            pltpu.SemaphoreType.DMA,
            pltpu.SemaphoreType.DMA,
        ],
    )
    def sc_kernel(table_hbm, i0_hbm, i1_hbm, i2_hbm, i3_hbm,
                  w0_hbm, w1_hbm, w2_hbm, w3_hbm, out_hbm,
                  idx_v, w_v, rows_v, out_v,
                  sem_io0, sem_io1, sem_g0, sem_g1, sem_o0, sem_o1):
        sem_io = [sem_io0, sem_io1]
        sem_g = [sem_g0, sem_g1]
        sem_o = [sem_o0, sem_o1]
        wid = lax.axis_index("s") * 2 + lax.axis_index("c")
        base0 = wid * _RPW
        ihs = [i0_hbm, i1_hbm, i2_hbm, i3_hbm]
        whs = [w0_hbm, w1_hbm, w2_hbm, w3_hbm]

        def load_idx(ci, b):
            o16 = (base0 + ci * _CH) * (NL * NP)
            for c in range(NCORN):
                pltpu.async_copy(ihs[c].at[pl.ds(o16, _IPC)],
                                 idx_v.at[b, pl.ds(c * _IPC, _IPC)], sem_io[b])
                pltpu.async_copy(whs[c].at[pl.ds(o16, _IPC)],
                                 w_v.at[b, pl.ds(c * _IPC, _IPC)], sem_io[b])

        def drain_idx(b):
            pltpu.make_async_copy(i0_hbm.at[pl.ds(0, NCORN * _IPC)],
                                  idx_v.at[b], sem_io[b]).wait()
            pltpu.make_async_copy(w0_hbm.at[pl.ds(0, NCORN * _IPC)],
                                  w_v.at[b], sem_io[b]).wait()

        def fire_gathers(b):
            for c in range(NCORN):
                for (s, n) in _SUBS:
                    pltpu.async_copy(
                        table_hbm.at[idx_v.at[b].at[pl.ds(c * _IPC + s, n)]],
                        rows_v.at[b].at[pl.ds(c * _IPC + s, n)], sem_g[b])

        def drain_gathers(b):
            pltpu.make_async_copy(table_hbm.at[pl.ds(0, NCORN * _IPC)],
                                  rows_v.at[b], sem_g[b]).wait()

        def compute(ci, b):
            @pl.loop(0, _CH)
            def _row(r):
                lin0 = r * (NL * NP)
                accs = []
                for c in range(NCORN):
                    a0 = jnp.zeros((16,), jnp.float32)
                    a1 = jnp.zeros((16,), jnp.float32)
                    wv = w_v[b, pl.ds(c * _IPC + lin0, 16)]
                    for u in range(16):
                        lin = c * _IPC + lin0 + u
                        wj = wv[u]
                        ev, od = plsc.unpack(rows_v[b, lin, :],
                                             format=plsc.PackFormat.INTERLEAVED)
                        a0 = a0 + wj * ev
                        a1 = a1 + wj * od
                    accs.append((a0, a1))
                acc0 = (accs[0][0] + accs[1][0]) + (accs[2][0] + accs[3][0])
                acc1 = (accs[0][1] + accs[1][1]) + (accs[2][1] + accs[3][1])
                out_v[b, pl.ds(r * HD, 16)] = acc0
                out_v[b, pl.ds(r * HD + 16, 16)] = acc1

            pltpu.async_copy(
                out_v.at[b],
                out_hbm.at[pl.ds((base0 + ci * _CH) * HD, _CH * HD)], sem_o[b])

        load_idx(0, 0)
        drain_idx(0)
        fire_gathers(0)
        load_idx(1, 1)

        @pl.loop(0, _NCHUNK, step=2)
        def _pipe(ci):
            for b in (0, 1):
                cur = ci + b
                nb = 1 - b
                drain_gathers(b)

                @pl.when(cur + 1 < _NCHUNK)
                def _():
                    drain_idx(nb)
                    fire_gathers(nb)

                @pl.when(cur >= 2)
                def _():
                    pltpu.make_async_copy(
                        out_v.at[b], out_hbm.at[pl.ds(0, _CH * HD)],
                        sem_o[b]).wait()

                compute(cur, b)

                @pl.when(cur + 2 < _NCHUNK)
                def _():
                    load_idx(cur + 2, b)

        for b in (0, 1):
            pltpu.make_async_copy(out_v.at[b],
                                  out_hbm.at[pl.ds(0, _CH * HD)],
                                  sem_o[b]).wait()

    return sc_kernel(table, *idxs, *wgts)


def kernel(query, reference_points, value, value_spatial_shapes,
           value_level_start_index, Wv, bv, Woff, boff, Wattn, battn,
           Wout, bout):
    L = NH * NL * NP

    v2 = _vproj(value.reshape(BS * LV, D), Wv, bv)
    table = v2.reshape(BS * LV * NH, HD)

    q2d = query.reshape(BQ, D)
    Wox = Woff[:, 0::2]
    Woy = Woff[:, 1::2]
    box = boff[0::2]
    boy = boff[1::2]
    rpx = jnp.broadcast_to(reference_points[..., 0].reshape(BQ, 1, NL, 1),
                           (BQ, NH, NL, NP)).reshape(BQ, L)
    rpy = jnp.broadcast_to(reference_points[..., 1].reshape(BQ, 1, NL, 1),
                           (BQ, NH, NL, NP)).reshape(BQ, L)
    brow = jnp.broadcast_to(
        (jnp.repeat(jnp.arange(BS, dtype=jnp.int32) * (LV * NH), LQ)
         ).reshape(BQ, 1), (BQ, L))
    i00, i10, i01, i11, w00, w10, w01, w11 = _params(
        q2d, Wox, Woy, box, boy, Wattn, battn, rpx, rpy, brow)

    idxs = [a.reshape(-1) for a in (i00, i10, i01, i11)]
    wgts = [a.reshape(-1) for a in (w00, w10, w01, w11)]

    sampled = _sc_sample(table, idxs, wgts)

    perm32 = np.concatenate([np.arange(0, HD, 2), np.arange(1, HD, 2)])
    permg = (np.arange(D) // HD) * HD + perm32[np.arange(D) % HD]
    out = _outproj(sampled.reshape(BQ, D), Wout[jnp.asarray(permg), :], bout)
    return out.reshape(BS, LQ, D)
